# baseline (speedup 1.0000x reference)
import functools

import jax
import jax.numpy as jnp
from jax import lax
from jax.experimental import pallas as pl
from jax.experimental.pallas import tpu as pltpu
from jax.experimental.pallas import tpu_sc as plsc

B = 16384
D = 32
NC = 2
NS = 16
NW = NC * NS
BPW = B // NW
CHUNK = 128
NCHUNK = BPW // CHUNK


def _dot_body(trip_hbm, table_hbm, out_hbm, trip_v, idx_v, left_v, right_v,
              out_v, sem):
    wid = lax.axis_index("s") * NC + lax.axis_index("c")
    base = wid * BPW

    pltpu.sync_copy(trip_hbm.at[pl.ds(base * 3, BPW * 3)], trip_v)

    iota = lax.iota(jnp.int32, 16)
    iota3 = iota * 3

    for k in range(NCHUNK):
        for jj in range(CHUNK // 16):
            fbase = iota3 + (k * CHUNK + jj * 16) * 3
            idx_v[k, pl.ds(jj * 16, 16)] = plsc.load_gather(
                trip_v, [fbase])
            idx_v[NCHUNK + k, pl.ds(jj * 16, 16)] = plsc.load_gather(
                trip_v, [fbase + 2])

    copies = []
    for k in range(NCHUNK):
        copies.append(pltpu.async_copy(
            table_hbm.at[idx_v.at[k]],
            left_v.at[pl.ds(k * CHUNK, CHUNK)], sem))
        copies.append(pltpu.async_copy(
            table_hbm.at[idx_v.at[NCHUNK + k]],
            right_v.at[pl.ds(k * CHUNK, CHUNK)], sem))
    for c in copies:
        c.wait()

    def block(j, carry):
        rows = iota + j * 16
        acc = jnp.zeros((16,), jnp.float32)
        for d in range(D):
            cold = jnp.full((16,), d, jnp.int32)
            l = plsc.load_gather(left_v, [rows, cold])
            r = plsc.load_gather(right_v, [rows, cold])
            acc = acc + l * r
        out_v[pl.ds(j * 16, 16)] = acc
        return carry

    lax.fori_loop(0, BPW // 16, block, 0)

    pltpu.sync_copy(out_v, out_hbm.at[pl.ds(base, BPW)])


def kernel(triplets, node_emb, vars):
    del vars
    mesh = plsc.VectorSubcoreMesh(core_axis_name="c", subcore_axis_name="s")
    f = functools.partial(
        pl.kernel,
        out_type=jax.ShapeDtypeStruct((B,), jnp.float32),
        mesh=mesh,
        compiler_params=pltpu.CompilerParams(
            needs_layout_passes=False, use_tc_tiling_on_sc=False),
        scratch_types=[
            pltpu.VMEM((BPW * 3,), jnp.int32),
            pltpu.VMEM((2 * NCHUNK, CHUNK), jnp.int32),
            pltpu.VMEM((BPW, D), jnp.float32),
            pltpu.VMEM((BPW, D), jnp.float32),
            pltpu.VMEM((BPW,), jnp.float32),
            pltpu.SemaphoreType.DMA,
        ],
    )(_dot_body)
    return f(triplets.reshape(-1), node_emb)
